# baseline (speedup 1.0000x reference)
import functools

import jax
import jax.numpy as jnp
from jax import lax
from jax.experimental import pallas as pl
from jax.experimental.pallas import tpu as pltpu
from jax.experimental.pallas import tpu_sc as plsc

_T = 2048
_H = 1024
_E = 64
_EG = 8
_NG = 8
_TKG = 4
_I = 512
_SI = 1024
_SCALE = 2.5
_BLK = 64
_G = _E + (2 * _T) // _BLK
_NPAD = _G * _BLK
_NEG = -1e30

_NC = 2
_NS = 16
_NW = _NC * _NS
_TPW = _T // _NW
_CH = 32


def _sigmoid(x):
    return 1.0 / (1.0 + jnp.exp(-x))


def _router_body(scores_ref, bias_ref, s0_ref, s1_ref, w0_ref, w1_ref,
                 eid_ref, geff_ref):
    f32 = jnp.float32
    i32 = jnp.int32
    scores = scores_ref[...]
    sc = scores + bias_ref[...]
    iota_e = lax.broadcasted_iota(i32, (_T, _E), 1)
    grp_of = lax.div(iota_e, _EG)

    gsums = []
    for g in range(_NG):
        v = jnp.where(grp_of == g, sc, _NEG)
        m1 = jnp.max(v, axis=1, keepdims=True)
        fi = jnp.min(jnp.where(v == m1, iota_e, _E), axis=1, keepdims=True)
        m2 = jnp.max(jnp.where(iota_e == fi, _NEG, v), axis=1, keepdims=True)
        gsums.append(m1 + m2)
    gs = jnp.concatenate(gsums, axis=1)

    iota_g = lax.broadcasted_iota(i32, (_T, _NG), 1)
    gmask = jnp.zeros((_T, _NG), f32)
    v = gs
    for _ in range(_TKG):
        m = jnp.max(v, axis=1, keepdims=True)
        fi = jnp.min(jnp.where(v == m, iota_g, _NG), axis=1, keepdims=True)
        hit = iota_g == fi
        gmask = gmask + hit.astype(f32)
        v = jnp.where(hit, _NEG, v)

    rep = (lax.broadcasted_iota(i32, (_NG, _E), 0)
           == lax.div(lax.broadcasted_iota(i32, (_NG, _E), 1), _EG)).astype(f32)
    score_mask = lax.dot_general(gmask, rep, (((1,), (0,)), ((), ())),
                                 preferred_element_type=f32)
    scc = jnp.where(score_mask > 0.5, sc, 0.0)

    ohs = []
    v = scc
    for _ in range(2):
        m = jnp.max(v, axis=1, keepdims=True)
        fi = jnp.min(jnp.where(v == m, iota_e, _E), axis=1, keepdims=True)
        hit = iota_e == fi
        ohs.append(hit.astype(f32))
        v = jnp.where(hit, _NEG, v)
    oh0, oh1 = ohs

    w0 = jnp.sum(oh0 * scores, axis=1, keepdims=True)
    w1 = jnp.sum(oh1 * scores, axis=1, keepdims=True)
    norm = w0 + w1 + 1e-20
    w0_ref[...] = w0 / norm * _SCALE
    w1_ref[...] = w1 / norm * _SCALE

    P = oh0 + oh1
    tri = (lax.broadcasted_iota(i32, (_T, _T), 0)
           > lax.broadcasted_iota(i32, (_T, _T), 1)).astype(f32)
    prefix = lax.dot_general(tri, P, (((1,), (0,)), ((), ())),
                             preferred_element_type=f32)
    counts = jnp.sum(P, axis=0, keepdims=True)
    blocks = jnp.floor((counts + (_BLK - 1)) * (1.0 / _BLK))
    u_strict = (lax.broadcasted_iota(i32, (_E, _E), 0)
                < lax.broadcasted_iota(i32, (_E, _E), 1)).astype(f32)
    u_incl = (lax.broadcasted_iota(i32, (_E, _E), 0)
              <= lax.broadcasted_iota(i32, (_E, _E), 1)).astype(f32)
    cum_excl = lax.dot_general(blocks, u_strict, (((1,), (0,)), ((), ())),
                               preferred_element_type=f32)
    cum_incl = lax.dot_general(blocks, u_incl, (((1,), (0,)), ((), ())),
                               preferred_element_type=f32)
    base = cum_excl * _BLK
    s0_ref[...] = jnp.sum(oh0 * (base + prefix), axis=1,
                          keepdims=True).astype(i32)
    s1_ref[...] = jnp.sum(oh1 * (base + prefix), axis=1,
                          keepdims=True).astype(i32)

    total = jnp.max(cum_incl, axis=1, keepdims=True)
    gcol = lax.broadcasted_iota(i32, (_G, 1), 0).astype(f32)
    geff = jnp.minimum(gcol, total - 1.0)
    eid = jnp.sum((cum_incl <= geff).astype(i32), axis=1, keepdims=True)
    eid_ref[...] = eid
    geff_ref[...] = geff.astype(i32)


def _run_router(scores, bias_row):
    i32 = jnp.int32
    f32 = jnp.float32
    outs = pl.pallas_call(
        _router_body,
        out_shape=(
            jax.ShapeDtypeStruct((_T, 1), i32),
            jax.ShapeDtypeStruct((_T, 1), i32),
            jax.ShapeDtypeStruct((_T, 1), f32),
            jax.ShapeDtypeStruct((_T, 1), f32),
            jax.ShapeDtypeStruct((_G, 1), i32),
            jax.ShapeDtypeStruct((_G, 1), i32),
        ),
    )(scores, bias_row)
    return outs


@functools.cache
def _get_dispatch():
    mesh = plsc.VectorSubcoreMesh(core_axis_name="c", subcore_axis_name="s")

    @functools.partial(
        pl.kernel,
        out_type=jax.ShapeDtypeStruct((_NPAD, _H), jnp.float32),
        mesh=mesh,
        scratch_types=[
            pltpu.VMEM((_TPW, _H), jnp.float32),
            pltpu.VMEM((_TPW,), jnp.int32),
            pltpu.VMEM((_TPW,), jnp.int32),
            pltpu.SemaphoreType.DMA,
        ],
    )
    def _dispatch(x_hbm, s0_hbm, s1_hbm, out_hbm, xrows, idx0, idx1, sem):
        wid = lax.axis_index("s") * _NC + lax.axis_index("c")
        base = wid * _TPW
        pltpu.sync_copy(s0_hbm.at[pl.ds(base, _TPW)], idx0)
        pltpu.sync_copy(s1_hbm.at[pl.ds(base, _TPW)], idx1)
        pltpu.sync_copy(x_hbm.at[pl.ds(base, _TPW)], xrows)
        c0 = pltpu.async_copy(xrows, out_hbm.at[idx0], sem)
        c1 = pltpu.async_copy(xrows, out_hbm.at[idx1], sem)
        c0.wait()
        c1.wait()

    return _dispatch


def _mlp_body(eid_ref, geff_ref, x_ref, gup_ref, dp_ref, sgw_ref, suw_ref,
              sdw_ref, y_ref, y2_ref):
    del eid_ref
    f32 = jnp.float32
    g = pl.program_id(0)

    @pl.when(g == geff_ref[g])
    def _():
        x = x_ref[...]
        gup = gup_ref[0]
        gu = lax.dot_general(x, gup, (((1,), (1,)), ((), ())),
                             preferred_element_type=f32)
        gate = gu[:, :_I]
        up = gu[:, _I:]
        h = gate * _sigmoid(gate) * up
        dp = dp_ref[0]
        y_ref[...] = lax.dot_general(h, dp, (((1,), (1,)), ((), ())),
                                     preferred_element_type=f32)
        sg = lax.dot_general(x, sgw_ref[...], (((1,), (1,)), ((), ())),
                             preferred_element_type=f32)
        su = lax.dot_general(x, suw_ref[...], (((1,), (1,)), ((), ())),
                             preferred_element_type=f32)
        ss = sg * _sigmoid(sg) * su
        y2_ref[...] = lax.dot_general(ss, sdw_ref[...], (((1,), (1,)), ((), ())),
                                      preferred_element_type=f32)


def _run_mlp(eid, geff, xs, gate_up_proj, down_proj, sgw, suw, sdw):
    grid_spec = pltpu.PrefetchScalarGridSpec(
        num_scalar_prefetch=2,
        grid=(_G,),
        in_specs=[
            pl.BlockSpec((_BLK, _H), lambda g, eid, geff: (geff[g], 0)),
            pl.BlockSpec((1, 2 * _I, _H), lambda g, eid, geff: (eid[g], 0, 0)),
            pl.BlockSpec((1, _H, _I), lambda g, eid, geff: (eid[g], 0, 0)),
            pl.BlockSpec((_SI, _H), lambda g, eid, geff: (0, 0)),
            pl.BlockSpec((_SI, _H), lambda g, eid, geff: (0, 0)),
            pl.BlockSpec((_H, _SI), lambda g, eid, geff: (0, 0)),
        ],
        out_specs=[
            pl.BlockSpec((_BLK, _H), lambda g, eid, geff: (geff[g], 0)),
            pl.BlockSpec((_BLK, _H), lambda g, eid, geff: (geff[g], 0)),
        ],
    )
    return pl.pallas_call(
        _mlp_body,
        grid_spec=grid_spec,
        out_shape=[
            jax.ShapeDtypeStruct((_NPAD, _H), jnp.float32),
            jax.ShapeDtypeStruct((_NPAD, _H), jnp.float32),
        ],
    )(eid, geff, xs, gate_up_proj, down_proj, sgw, suw, sdw)


@functools.cache
def _get_combine():
    mesh = plsc.VectorSubcoreMesh(core_axis_name="c", subcore_axis_name="s")

    @functools.partial(
        pl.kernel,
        out_type=jax.ShapeDtypeStruct((_T, _H), jnp.float32),
        mesh=mesh,
        scratch_types=[
            pltpu.VMEM((_CH, _H), jnp.float32),
            pltpu.VMEM((_CH, _H), jnp.float32),
            pltpu.VMEM((_CH, _H), jnp.float32),
            pltpu.VMEM((_CH,), jnp.int32),
            pltpu.VMEM((_CH,), jnp.int32),
            pltpu.VMEM((_CH + 16,), jnp.float32),
            pltpu.VMEM((_CH + 16,), jnp.float32),
            pltpu.SemaphoreType.DMA,
        ],
    )
    def _combine(y_hbm, y2_hbm, s0_hbm, s1_hbm, w0_hbm, w1_hbm, out_hbm,
                 y0_v, y1_v, sh_v, i0_v, i1_v, w0_v, w1_v, sem):
        wid = lax.axis_index("s") * _NC + lax.axis_index("c")
        i32 = jnp.int32

        for c in range(_TPW // _CH):
            tok0 = wid * _TPW + c * _CH
            pltpu.sync_copy(s0_hbm.at[pl.ds(tok0, _CH)], i0_v)
            pltpu.sync_copy(s1_hbm.at[pl.ds(tok0, _CH)], i1_v)
            pltpu.sync_copy(w0_hbm.at[pl.ds(tok0, _CH)], w0_v.at[pl.ds(0, _CH)])
            pltpu.sync_copy(w1_hbm.at[pl.ds(tok0, _CH)], w1_v.at[pl.ds(0, _CH)])
            g0 = pltpu.async_copy(y_hbm.at[i0_v], y0_v, sem)
            g1 = pltpu.async_copy(y_hbm.at[i1_v], y1_v, sem)
            g2 = pltpu.async_copy(y2_hbm.at[i0_v], sh_v, sem)
            g0.wait()
            g1.wait()
            g2.wait()

            def tok_body(i, carry):
                w0s = jnp.full((16,), w0_v[pl.ds(i, 16)][0], jnp.float32)
                w1s = jnp.full((16,), w1_v[pl.ds(i, 16)][0], jnp.float32)
                for cc in range(_H // 16):
                    sl = pl.ds(cc * 16, 16)
                    a = y0_v[i, sl]
                    b = y1_v[i, sl]
                    sh = sh_v[i, sl]
                    sh_v[i, sl] = w0s * a + w1s * b + sh
                return carry

            lax.fori_loop(0, _CH, tok_body, 0)
            pltpu.sync_copy(sh_v, out_hbm.at[pl.ds(tok0, _CH)])

    return _combine


def kernel(hidden_states, gate_up_proj, down_proj, router_weight, e_score_bias,
           shared_gate_w, shared_up_w, shared_down_w):
    x2d = hidden_states.reshape(-1, _H)
    logits = x2d.astype(jnp.float32) @ router_weight.astype(jnp.float32).T
    scores = jax.nn.sigmoid(logits)

    s0, s1, w0, w1, eid, geff = _run_router(scores, e_score_bias.reshape(1, _E))
    s0 = s0.reshape(_T)
    s1 = s1.reshape(_T)
    w0 = w0.reshape(_T)
    w1 = w1.reshape(_T)
    eid = eid.reshape(_G)
    geff = geff.reshape(_G)

    xs = _get_dispatch()(x2d, s0, s1)
    y, y2 = _run_mlp(eid, geff, xs, gate_up_proj, down_proj,
                     shared_gate_w, shared_up_w, shared_down_w)
    out = _get_combine()(y, y2, s0, s1, w0, w1)
    return out.reshape(hidden_states.shape)


# speedup vs baseline: 1.2427x; 1.2427x over previous
import functools

import jax
import jax.numpy as jnp
from jax import lax
from jax.experimental import pallas as pl
from jax.experimental.pallas import tpu as pltpu
from jax.experimental.pallas import tpu_sc as plsc

_T = 2048
_H = 1024
_E = 64
_EG = 8
_NG = 8
_TKG = 4
_I = 512
_SI = 1024
_SCALE = 2.5
_BLK = 64
_G = _E + (2 * _T) // _BLK
_NPAD = _G * _BLK
_NEG = -1e30

_NC = 2
_NS = 16
_NW = _NC * _NS
_TPW = _T // _NW
_CH = 32


def _sigmoid(x):
    return 1.0 / (1.0 + jnp.exp(-x))


def _router_body(scores_ref, bias_ref, s0_ref, s1_ref, w0_ref, w1_ref,
                 eid_ref, geff_ref):
    f32 = jnp.float32
    i32 = jnp.int32
    scores = scores_ref[...]
    sc = scores + bias_ref[...]
    iota_e = lax.broadcasted_iota(i32, (_T, _E), 1)
    iota_sub = lax.broadcasted_iota(i32, (_T, _EG), 1)

    gsums = []
    for g in range(_NG):
        v = sc[:, g * _EG:(g + 1) * _EG]
        m1 = jnp.max(v, axis=1, keepdims=True)
        fi = jnp.min(jnp.where(v == m1, iota_sub, _EG), axis=1, keepdims=True)
        m2 = jnp.max(jnp.where(iota_sub == fi, _NEG, v), axis=1, keepdims=True)
        gsums.append(m1 + m2)
    gs = jnp.concatenate(gsums, axis=1)

    iota_g = lax.broadcasted_iota(i32, (_T, _NG), 1)
    gmask = jnp.zeros((_T, _NG), f32)
    v = gs
    for _ in range(_TKG):
        m = jnp.max(v, axis=1, keepdims=True)
        fi = jnp.min(jnp.where(v == m, iota_g, _NG), axis=1, keepdims=True)
        hit = iota_g == fi
        gmask = gmask + hit.astype(f32)
        v = jnp.where(hit, _NEG, v)

    rep = (lax.broadcasted_iota(i32, (_NG, _E), 0)
           == lax.div(lax.broadcasted_iota(i32, (_NG, _E), 1), _EG)).astype(f32)
    score_mask = lax.dot_general(gmask, rep, (((1,), (0,)), ((), ())),
                                 preferred_element_type=f32)
    scc = jnp.where(score_mask > 0.5, sc, 0.0)

    ohs = []
    v = scc
    for _ in range(2):
        m = jnp.max(v, axis=1, keepdims=True)
        fi = jnp.min(jnp.where(v == m, iota_e, _E), axis=1, keepdims=True)
        hit = iota_e == fi
        ohs.append(hit.astype(f32))
        v = jnp.where(hit, _NEG, v)
    oh0, oh1 = ohs

    w0 = jnp.sum(oh0 * scores, axis=1, keepdims=True)
    w1 = jnp.sum(oh1 * scores, axis=1, keepdims=True)
    norm = w0 + w1 + 1e-20
    w0_ref[...] = w0 / norm * _SCALE
    w1_ref[...] = w1 / norm * _SCALE

    P = oh0 + oh1
    _C = 128
    tri_c = (lax.broadcasted_iota(i32, (_C, _C), 0)
             > lax.broadcasted_iota(i32, (_C, _C), 1)).astype(f32)
    parts = []
    running = jnp.zeros((1, _E), f32)
    for c in range(_T // _C):
        Pc = P[c * _C:(c + 1) * _C, :]
        local = lax.dot_general(tri_c, Pc, (((1,), (0,)), ((), ())),
                                preferred_element_type=f32)
        parts.append(local + running)
        running = running + jnp.sum(Pc, axis=0, keepdims=True)
    prefix = jnp.concatenate(parts, axis=0)
    counts = jnp.sum(P, axis=0, keepdims=True)
    blocks = jnp.floor((counts + (_BLK - 1)) * (1.0 / _BLK))
    u_strict = (lax.broadcasted_iota(i32, (_E, _E), 0)
                < lax.broadcasted_iota(i32, (_E, _E), 1)).astype(f32)
    u_incl = (lax.broadcasted_iota(i32, (_E, _E), 0)
              <= lax.broadcasted_iota(i32, (_E, _E), 1)).astype(f32)
    cum_excl = lax.dot_general(blocks, u_strict, (((1,), (0,)), ((), ())),
                               preferred_element_type=f32)
    cum_incl = lax.dot_general(blocks, u_incl, (((1,), (0,)), ((), ())),
                               preferred_element_type=f32)
    base = cum_excl * _BLK
    s0_ref[...] = jnp.sum(oh0 * (base + prefix), axis=1,
                          keepdims=True).astype(i32)
    s1_ref[...] = jnp.sum(oh1 * (base + prefix), axis=1,
                          keepdims=True).astype(i32)

    total = jnp.max(cum_incl, axis=1, keepdims=True)
    gcol = lax.broadcasted_iota(i32, (_G, 1), 0).astype(f32)
    geff = jnp.minimum(gcol, total - 1.0)
    eid = jnp.sum((cum_incl <= geff).astype(i32), axis=1, keepdims=True)
    eid_ref[...] = eid
    geff_ref[...] = geff.astype(i32)


def _run_router(scores, bias_row):
    i32 = jnp.int32
    f32 = jnp.float32
    outs = pl.pallas_call(
        _router_body,
        out_shape=(
            jax.ShapeDtypeStruct((_T, 1), i32),
            jax.ShapeDtypeStruct((_T, 1), i32),
            jax.ShapeDtypeStruct((_T, 1), f32),
            jax.ShapeDtypeStruct((_T, 1), f32),
            jax.ShapeDtypeStruct((_G, 1), i32),
            jax.ShapeDtypeStruct((_G, 1), i32),
        ),
    )(scores, bias_row)
    return outs


@functools.cache
def _get_dispatch():
    mesh = plsc.VectorSubcoreMesh(core_axis_name="c", subcore_axis_name="s")

    @functools.partial(
        pl.kernel,
        out_type=jax.ShapeDtypeStruct((_NPAD, _H), jnp.float32),
        mesh=mesh,
        scratch_types=[
            pltpu.VMEM((_TPW, _H), jnp.float32),
            pltpu.VMEM((_TPW,), jnp.int32),
            pltpu.VMEM((_TPW,), jnp.int32),
            pltpu.SemaphoreType.DMA,
        ],
    )
    def _dispatch(x_hbm, s0_hbm, s1_hbm, out_hbm, xrows, idx0, idx1, sem):
        wid = lax.axis_index("s") * _NC + lax.axis_index("c")
        base = wid * _TPW
        pltpu.sync_copy(s0_hbm.at[pl.ds(base, _TPW)], idx0)
        pltpu.sync_copy(s1_hbm.at[pl.ds(base, _TPW)], idx1)
        pltpu.sync_copy(x_hbm.at[pl.ds(base, _TPW)], xrows)
        c0 = pltpu.async_copy(xrows, out_hbm.at[idx0], sem)
        c1 = pltpu.async_copy(xrows, out_hbm.at[idx1], sem)
        c0.wait()
        c1.wait()

    return _dispatch


def _mlp_body(eid_ref, geff_ref, x_ref, gup_ref, dp_ref, y_ref):
    del eid_ref
    f32 = jnp.float32
    g = pl.program_id(0)

    @pl.when(g == geff_ref[g])
    def _():
        x = x_ref[...]
        gup = gup_ref[0]
        gu = lax.dot_general(x, gup, (((1,), (1,)), ((), ())),
                             preferred_element_type=f32)
        gate = gu[:, :_I]
        up = gu[:, _I:]
        h = gate * _sigmoid(gate) * up
        dp = dp_ref[0]
        y_ref[...] = lax.dot_general(h, dp, (((1,), (1,)), ((), ())),
                                     preferred_element_type=f32)


def _run_mlp(eid, geff, xs, gate_up_proj, down_proj):
    grid_spec = pltpu.PrefetchScalarGridSpec(
        num_scalar_prefetch=2,
        grid=(_G,),
        in_specs=[
            pl.BlockSpec((_BLK, _H), lambda g, eid, geff: (geff[g], 0)),
            pl.BlockSpec((1, 2 * _I, _H), lambda g, eid, geff: (eid[g], 0, 0)),
            pl.BlockSpec((1, _H, _I), lambda g, eid, geff: (eid[g], 0, 0)),
        ],
        out_specs=pl.BlockSpec((_BLK, _H), lambda g, eid, geff: (geff[g], 0)),
    )
    return pl.pallas_call(
        _mlp_body,
        grid_spec=grid_spec,
        out_shape=jax.ShapeDtypeStruct((_NPAD, _H), jnp.float32),
    )(eid, geff, xs, gate_up_proj, down_proj)


def _shared_body(x_ref, gw_ref, uw_ref, dw_ref, o_ref):
    f32 = jnp.float32
    bf16 = jnp.bfloat16
    x = x_ref[...].astype(bf16)
    g = lax.dot_general(x, gw_ref[...].astype(bf16), (((1,), (1,)), ((), ())),
                        preferred_element_type=f32)
    u = lax.dot_general(x, uw_ref[...].astype(bf16), (((1,), (1,)), ((), ())),
                        preferred_element_type=f32)
    s = (g * _sigmoid(g) * u).astype(bf16)
    o_ref[...] = lax.dot_general(s, dw_ref[...].astype(bf16),
                                 (((1,), (1,)), ((), ())),
                                 preferred_element_type=f32)


def _run_shared(x2d, gw, uw, dw):
    tb = 1024
    return pl.pallas_call(
        _shared_body,
        grid=(_T // tb,),
        in_specs=[
            pl.BlockSpec((tb, _H), lambda i: (i, 0)),
            pl.BlockSpec((_SI, _H), lambda i: (0, 0)),
            pl.BlockSpec((_SI, _H), lambda i: (0, 0)),
            pl.BlockSpec((_H, _SI), lambda i: (0, 0)),
        ],
        out_specs=pl.BlockSpec((tb, _H), lambda i: (i, 0)),
        out_shape=jax.ShapeDtypeStruct((_T, _H), jnp.float32),
    )(x2d, gw, uw, dw)


@functools.cache
def _get_combine():
    mesh = plsc.VectorSubcoreMesh(core_axis_name="c", subcore_axis_name="s")

    @functools.partial(
        pl.kernel,
        out_type=jax.ShapeDtypeStruct((_T, _H), jnp.float32),
        mesh=mesh,
        scratch_types=[
            pltpu.VMEM((_CH, _H), jnp.float32),
            pltpu.VMEM((_CH, _H), jnp.float32),
            pltpu.VMEM((_CH, _H), jnp.float32),
            pltpu.VMEM((_CH,), jnp.int32),
            pltpu.VMEM((_CH,), jnp.int32),
            pltpu.VMEM((_CH + 16,), jnp.float32),
            pltpu.VMEM((_CH + 16,), jnp.float32),
            pltpu.SemaphoreType.DMA,
        ],
    )
    def _combine(y_hbm, sh_hbm, s0_hbm, s1_hbm, w0_hbm, w1_hbm, out_hbm,
                 y0_v, y1_v, sh_v, i0_v, i1_v, w0_v, w1_v, sem):
        wid = lax.axis_index("s") * _NC + lax.axis_index("c")
        i32 = jnp.int32

        for c in range(_TPW // _CH):
            tok0 = wid * _TPW + c * _CH
            pltpu.sync_copy(s0_hbm.at[pl.ds(tok0, _CH)], i0_v)
            pltpu.sync_copy(s1_hbm.at[pl.ds(tok0, _CH)], i1_v)
            pltpu.sync_copy(w0_hbm.at[pl.ds(tok0, _CH)], w0_v.at[pl.ds(0, _CH)])
            pltpu.sync_copy(w1_hbm.at[pl.ds(tok0, _CH)], w1_v.at[pl.ds(0, _CH)])
            g0 = pltpu.async_copy(y_hbm.at[i0_v], y0_v, sem)
            g1 = pltpu.async_copy(y_hbm.at[i1_v], y1_v, sem)
            pltpu.sync_copy(sh_hbm.at[pl.ds(tok0, _CH)], sh_v)
            g0.wait()
            g1.wait()

            def tok_body(i, carry):
                w0s = jnp.full((16,), w0_v[pl.ds(i, 16)][0], jnp.float32)
                w1s = jnp.full((16,), w1_v[pl.ds(i, 16)][0], jnp.float32)
                for cc in range(_H // 16):
                    sl = pl.ds(cc * 16, 16)
                    a = y0_v[i, sl]
                    b = y1_v[i, sl]
                    sh = sh_v[i, sl]
                    sh_v[i, sl] = w0s * a + w1s * b + sh
                return carry

            lax.fori_loop(0, _CH, tok_body, 0)
            pltpu.sync_copy(sh_v, out_hbm.at[pl.ds(tok0, _CH)])

    return _combine


def kernel(hidden_states, gate_up_proj, down_proj, router_weight, e_score_bias,
           shared_gate_w, shared_up_w, shared_down_w):
    x2d = hidden_states.reshape(-1, _H)
    logits = x2d.astype(jnp.float32) @ router_weight.astype(jnp.float32).T
    scores = jax.nn.sigmoid(logits)

    s0, s1, w0, w1, eid, geff = _run_router(scores, e_score_bias.reshape(1, _E))
    s0 = s0.reshape(_T)
    s1 = s1.reshape(_T)
    w0 = w0.reshape(_T)
    w1 = w1.reshape(_T)
    eid = eid.reshape(_G)
    geff = geff.reshape(_G)

    xs = _get_dispatch()(x2d, s0, s1)
    y = _run_mlp(eid, geff, xs, gate_up_proj, down_proj)
    sh = _run_shared(x2d, shared_gate_w, shared_up_w, shared_down_w)
    out = _get_combine()(y, sh, s0, s1, w0, w1)
    return out.reshape(hidden_states.shape)


# speedup vs baseline: 1.2986x; 1.0450x over previous
import functools

import jax
import jax.numpy as jnp
from jax import lax
from jax.experimental import pallas as pl
from jax.experimental.pallas import tpu as pltpu
from jax.experimental.pallas import tpu_sc as plsc

_T = 2048
_H = 1024
_E = 64
_EG = 8
_NG = 8
_TKG = 4
_I = 512
_SI = 1024
_SCALE = 2.5
_BLK = 64
_G = _E + (2 * _T) // _BLK
_NPAD = _G * _BLK
_NEG = -1e30

_NC = 2
_NS = 16
_NW = _NC * _NS
_TPW = _T // _NW
_CH = 16


def _sigmoid(x):
    return 1.0 / (1.0 + jnp.exp(-x))


def _router_body(scores_ref, bias_ref, s0_ref, s1_ref, w0_ref, w1_ref,
                 eid_ref, geff_ref):
    f32 = jnp.float32
    i32 = jnp.int32
    scores = scores_ref[...]
    sc = scores + bias_ref[...]
    iota_e = lax.broadcasted_iota(i32, (_T, _E), 1)
    grp_of = lax.div(iota_e, _EG)

    gsums = []
    for g in range(_NG):
        v = jnp.where(grp_of == g, sc, _NEG)
        m1 = jnp.max(v, axis=1, keepdims=True)
        fi = jnp.min(jnp.where(v == m1, iota_e, _E), axis=1, keepdims=True)
        m2 = jnp.max(jnp.where(iota_e == fi, _NEG, v), axis=1, keepdims=True)
        gsums.append(m1 + m2)
    gs = jnp.concatenate(gsums, axis=1)

    iota_g = lax.broadcasted_iota(i32, (_T, _NG), 1)
    gmask = jnp.zeros((_T, _NG), f32)
    v = gs
    for _ in range(_TKG):
        m = jnp.max(v, axis=1, keepdims=True)
        fi = jnp.min(jnp.where(v == m, iota_g, _NG), axis=1, keepdims=True)
        hit = iota_g == fi
        gmask = gmask + hit.astype(f32)
        v = jnp.where(hit, _NEG, v)

    rep = (lax.broadcasted_iota(i32, (_NG, _E), 0)
           == lax.div(lax.broadcasted_iota(i32, (_NG, _E), 1), _EG)).astype(f32)
    score_mask = lax.dot_general(gmask, rep, (((1,), (0,)), ((), ())),
                                 preferred_element_type=f32)
    scc = jnp.where(score_mask > 0.5, sc, 0.0)

    ohs = []
    v = scc
    for _ in range(2):
        m = jnp.max(v, axis=1, keepdims=True)
        fi = jnp.min(jnp.where(v == m, iota_e, _E), axis=1, keepdims=True)
        hit = iota_e == fi
        ohs.append(hit.astype(f32))
        v = jnp.where(hit, _NEG, v)
    oh0, oh1 = ohs

    w0 = jnp.sum(oh0 * scores, axis=1, keepdims=True)
    w1 = jnp.sum(oh1 * scores, axis=1, keepdims=True)
    norm = w0 + w1 + 1e-20
    w0_ref[...] = w0 / norm * _SCALE
    w1_ref[...] = w1 / norm * _SCALE

    P = oh0 + oh1
    tri = (lax.broadcasted_iota(i32, (_T, _T), 0)
           > lax.broadcasted_iota(i32, (_T, _T), 1)).astype(jnp.bfloat16)
    prefix = lax.dot_general(tri, P.astype(jnp.bfloat16),
                             (((1,), (0,)), ((), ())),
                             preferred_element_type=f32)
    counts = jnp.sum(P, axis=0, keepdims=True)
    blocks = jnp.floor((counts + (_BLK - 1)) * (1.0 / _BLK))
    u_strict = (lax.broadcasted_iota(i32, (_E, _E), 0)
                < lax.broadcasted_iota(i32, (_E, _E), 1)).astype(f32)
    u_incl = (lax.broadcasted_iota(i32, (_E, _E), 0)
              <= lax.broadcasted_iota(i32, (_E, _E), 1)).astype(f32)
    cum_excl = lax.dot_general(blocks, u_strict, (((1,), (0,)), ((), ())),
                               preferred_element_type=f32)
    cum_incl = lax.dot_general(blocks, u_incl, (((1,), (0,)), ((), ())),
                               preferred_element_type=f32)
    base = cum_excl * _BLK
    s0_ref[...] = jnp.sum(oh0 * (base + prefix), axis=1,
                          keepdims=True).astype(i32)
    s1_ref[...] = jnp.sum(oh1 * (base + prefix), axis=1,
                          keepdims=True).astype(i32)

    total = jnp.max(cum_incl, axis=1, keepdims=True)
    gcol = lax.broadcasted_iota(i32, (_G, 1), 0).astype(f32)
    geff = jnp.minimum(gcol, total - 1.0)
    eid = jnp.sum((cum_incl <= geff).astype(i32), axis=1, keepdims=True)
    eid_ref[...] = eid
    geff_ref[...] = geff.astype(i32)


def _run_router(scores, bias_row):
    i32 = jnp.int32
    f32 = jnp.float32
    outs = pl.pallas_call(
        _router_body,
        out_shape=(
            jax.ShapeDtypeStruct((_T, 1), i32),
            jax.ShapeDtypeStruct((_T, 1), i32),
            jax.ShapeDtypeStruct((_T, 1), f32),
            jax.ShapeDtypeStruct((_T, 1), f32),
            jax.ShapeDtypeStruct((_G, 1), i32),
            jax.ShapeDtypeStruct((_G, 1), i32),
        ),
    )(scores, bias_row)
    return outs


@functools.cache
def _get_dispatch():
    mesh = plsc.VectorSubcoreMesh(core_axis_name="c", subcore_axis_name="s")

    @functools.partial(
        pl.kernel,
        out_type=jax.ShapeDtypeStruct((_NPAD, _H), jnp.float32),
        mesh=mesh,
        scratch_types=[
            pltpu.VMEM((_TPW, _H), jnp.float32),
            pltpu.VMEM((_TPW,), jnp.int32),
            pltpu.VMEM((_TPW,), jnp.int32),
            pltpu.SemaphoreType.DMA,
        ],
    )
    def _dispatch(x_hbm, s0_hbm, s1_hbm, out_hbm, xrows, idx0, idx1, sem):
        wid = lax.axis_index("s") * _NC + lax.axis_index("c")
        base = wid * _TPW
        pltpu.sync_copy(s0_hbm.at[pl.ds(base, _TPW)], idx0)
        pltpu.sync_copy(s1_hbm.at[pl.ds(base, _TPW)], idx1)
        pltpu.sync_copy(x_hbm.at[pl.ds(base, _TPW)], xrows)
        c0 = pltpu.async_copy(xrows, out_hbm.at[idx0], sem)
        c1 = pltpu.async_copy(xrows, out_hbm.at[idx1], sem)
        c0.wait()
        c1.wait()

    return _dispatch


def _mlp_body(eid_ref, geff_ref, x_ref, gup_ref, dp_ref, y_ref):
    del eid_ref
    f32 = jnp.float32
    g = pl.program_id(0)

    @pl.when(g == geff_ref[g])
    def _():
        x = x_ref[...]
        gup = gup_ref[0]
        gu = lax.dot_general(x, gup, (((1,), (1,)), ((), ())),
                             preferred_element_type=f32)
        gate = gu[:, :_I]
        up = gu[:, _I:]
        h = gate * _sigmoid(gate) * up
        dp = dp_ref[0]
        y_ref[...] = lax.dot_general(h, dp, (((1,), (1,)), ((), ())),
                                     preferred_element_type=f32)


def _run_mlp(eid, geff, xs, gate_up_proj, down_proj):
    grid_spec = pltpu.PrefetchScalarGridSpec(
        num_scalar_prefetch=2,
        grid=(_G,),
        in_specs=[
            pl.BlockSpec((_BLK, _H), lambda g, eid, geff: (geff[g], 0)),
            pl.BlockSpec((1, 2 * _I, _H), lambda g, eid, geff: (eid[g], 0, 0)),
            pl.BlockSpec((1, _H, _I), lambda g, eid, geff: (eid[g], 0, 0)),
        ],
        out_specs=pl.BlockSpec((_BLK, _H), lambda g, eid, geff: (geff[g], 0)),
    )
    return pl.pallas_call(
        _mlp_body,
        grid_spec=grid_spec,
        out_shape=jax.ShapeDtypeStruct((_NPAD, _H), jnp.float32),
    )(eid, geff, xs, gate_up_proj, down_proj)


def _shared_body(x_ref, gw_ref, uw_ref, dw_ref, o_ref):
    f32 = jnp.float32
    bf16 = jnp.bfloat16
    x = x_ref[...].astype(bf16)
    g = lax.dot_general(x, gw_ref[...].astype(bf16), (((1,), (1,)), ((), ())),
                        preferred_element_type=f32)
    u = lax.dot_general(x, uw_ref[...].astype(bf16), (((1,), (1,)), ((), ())),
                        preferred_element_type=f32)
    s = (g * _sigmoid(g) * u).astype(bf16)
    o_ref[...] = lax.dot_general(s, dw_ref[...].astype(bf16),
                                 (((1,), (1,)), ((), ())),
                                 preferred_element_type=f32)


def _run_shared(x2d, gw, uw, dw):
    tb = 1024
    return pl.pallas_call(
        _shared_body,
        grid=(_T // tb,),
        in_specs=[
            pl.BlockSpec((tb, _H), lambda i: (i, 0)),
            pl.BlockSpec((_SI, _H), lambda i: (0, 0)),
            pl.BlockSpec((_SI, _H), lambda i: (0, 0)),
            pl.BlockSpec((_H, _SI), lambda i: (0, 0)),
        ],
        out_specs=pl.BlockSpec((tb, _H), lambda i: (i, 0)),
        out_shape=jax.ShapeDtypeStruct((_T, _H), jnp.float32),
    )(x2d, gw, uw, dw)


_NCHUNK = _TPW // _CH


@functools.cache
def _get_combine():
    mesh = plsc.VectorSubcoreMesh(core_axis_name="c", subcore_axis_name="s")

    @functools.partial(
        pl.kernel,
        out_type=jax.ShapeDtypeStruct((_T, _H), jnp.float32),
        mesh=mesh,
        scratch_types=[
            [pltpu.VMEM((_CH, _H), jnp.float32)] * 2,
            [pltpu.VMEM((_CH, _H), jnp.float32)] * 2,
            [pltpu.VMEM((_CH, _H), jnp.float32)] * 2,
            pltpu.VMEM((_TPW,), jnp.int32),
            pltpu.VMEM((_TPW,), jnp.int32),
            pltpu.VMEM((_TPW + 16,), jnp.float32),
            pltpu.VMEM((_TPW + 16,), jnp.float32),
            [pltpu.SemaphoreType.DMA] * 2,
        ],
    )
    def _combine(y_hbm, sh_hbm, s0_hbm, s1_hbm, w0_hbm, w1_hbm, out_hbm,
                 y0_v, y1_v, sh_v, i0_v, i1_v, w0_v, w1_v, sem):
        wid = lax.axis_index("s") * _NC + lax.axis_index("c")
        tok0 = wid * _TPW

        pltpu.sync_copy(s0_hbm.at[pl.ds(tok0, _TPW)], i0_v)
        pltpu.sync_copy(s1_hbm.at[pl.ds(tok0, _TPW)], i1_v)
        pltpu.sync_copy(w0_hbm.at[pl.ds(tok0, _TPW)], w0_v.at[pl.ds(0, _TPW)])
        pltpu.sync_copy(w1_hbm.at[pl.ds(tok0, _TPW)], w1_v.at[pl.ds(0, _TPW)])

        def issue(c, b):
            csl = pl.ds(c * _CH, _CH)
            g0 = pltpu.async_copy(y_hbm.at[i0_v.at[csl]], y0_v[b], sem[b])
            g1 = pltpu.async_copy(y_hbm.at[i1_v.at[csl]], y1_v[b], sem[b])
            g2 = pltpu.async_copy(sh_hbm.at[pl.ds(tok0 + c * _CH, _CH)],
                                  sh_v[b], sem[b])
            return (g0, g1, g2)

        pend = issue(0, 0)
        for c in range(_NCHUNK):
            b = c % 2
            for g in pend:
                g.wait()
            if c + 1 < _NCHUNK:
                pend = issue(c + 1, 1 - b)

            def tok_body(i, carry):
                w0s = jnp.full((16,), w0_v[pl.ds(c * _CH + i, 16)][0],
                               jnp.float32)
                w1s = jnp.full((16,), w1_v[pl.ds(c * _CH + i, 16)][0],
                               jnp.float32)
                for cc in range(_H // 16):
                    sl = pl.ds(cc * 16, 16)
                    sh_v[b][i, sl] = (w0s * y0_v[b][i, sl]
                                      + w1s * y1_v[b][i, sl] + sh_v[b][i, sl])
                return carry

            lax.fori_loop(0, _CH, tok_body, 0)
            pltpu.sync_copy(sh_v[b], out_hbm.at[pl.ds(tok0 + c * _CH, _CH)])

    return _combine


def kernel(hidden_states, gate_up_proj, down_proj, router_weight, e_score_bias,
           shared_gate_w, shared_up_w, shared_down_w):
    x2d = hidden_states.reshape(-1, _H)
    logits = x2d.astype(jnp.float32) @ router_weight.astype(jnp.float32).T
    scores = jax.nn.sigmoid(logits)

    s0, s1, w0, w1, eid, geff = _run_router(scores, e_score_bias.reshape(1, _E))
    s0 = s0.reshape(_T)
    s1 = s1.reshape(_T)
    w0 = w0.reshape(_T)
    w1 = w1.reshape(_T)
    eid = eid.reshape(_G)
    geff = geff.reshape(_G)

    xs = _get_dispatch()(x2d, s0, s1)
    y = _run_mlp(eid, geff, xs, gate_up_proj, down_proj)
    sh = _run_shared(x2d, shared_gate_w, shared_up_w, shared_down_w)
    out = _get_combine()(y, sh, s0, s1, w0, w1)
    return out.reshape(hidden_states.shape)
